# Initial kernel scaffold; baseline (speedup 1.0000x reference)
import jax
import jax.numpy as jnp
from jax.experimental import pallas as pl


def kernel(x, edge_index, W1, b1, W2, b2):
    raise NotImplementedError("write your pallas kernel here")


# speedup vs baseline: 16.7702x; 16.7702x over previous
import functools

import jax
import jax.numpy as jnp
from jax import lax
from jax.experimental import pallas as pl
from jax.experimental.pallas import tpu as pltpu
from jax.experimental.pallas import tpu_sc as plsc

NC = 2
NS = 16
NW = NC * NS
D = 128
N_PAD = 10240
RPT = N_PAD // NS
K = 128
RB = 1024


def _sc_mesh():
    return plsc.VectorSubcoreMesh(
        core_axis_name="c", subcore_axis_name="s", num_cores=NC, num_subcores=NS
    )


def _chunk_range(wid, total_chunks):
    base = total_chunks // NW
    rem = total_chunks - base * NW
    start = wid * base + jnp.minimum(wid, rem)
    cnt = base + (wid < rem).astype(jnp.int32)
    return start, cnt


def _deg_body(dst_hbm, p0_hbm, p1_hbm, ones_v, dst_v, zbuf_v, deg_sh):
    cid = lax.axis_index("c")
    sid = lax.axis_index("s")
    wid = cid * NS + sid
    one16 = jnp.ones((16,), jnp.float32)
    z16 = jnp.zeros((16,), jnp.float32)
    for l in range(K // 16):
        ones_v[pl.ds(l * 16, 16)] = one16
        zbuf_v[pl.ds(l * 16, 16)] = z16
    for r in range(RPT // K):
        pltpu.sync_copy(zbuf_v, deg_sh.at[pl.ds(sid * RPT + r * K, K)])
    plsc.subcore_barrier()

    total_chunks = dst_hbm.shape[0] // K
    start, cnt = _chunk_range(wid, total_chunks)

    def body(i, carry):
        off = (start + i) * K
        pltpu.sync_copy(dst_hbm.at[pl.ds(off, K)], dst_v)
        pltpu.sync_copy(ones_v, deg_sh.at[dst_v], add=True)
        return carry

    lax.fori_loop(0, cnt, body, 0)
    plsc.subcore_barrier()

    row0 = sid * RPT

    @pl.when(cid == 0)
    def _():
        pltpu.sync_copy(deg_sh.at[pl.ds(row0, RPT)], p0_hbm.at[pl.ds(row0, RPT)])

    @pl.when(cid == 1)
    def _():
        pltpu.sync_copy(deg_sh.at[pl.ds(row0, RPT)], p1_hbm.at[pl.ds(row0, RPT)])


def _deg_call(dst):
    f = pl.kernel(
        _deg_body,
        out_type=(
            jax.ShapeDtypeStruct((N_PAD,), jnp.float32),
            jax.ShapeDtypeStruct((N_PAD,), jnp.float32),
        ),
        mesh=_sc_mesh(),
        scratch_types=[
            pltpu.VMEM((K,), jnp.float32),
            pltpu.VMEM((K,), jnp.int32),
            pltpu.VMEM((K,), jnp.float32),
            pltpu.VMEM_SHARED((N_PAD,), jnp.float32),
        ],
    )
    return f(dst)


def _agg_body(y_hbm, src_hbm, dst_hbm, p0_hbm, p1_hbm,
              src_v, dst_v, rows_v, acc_sh, sem):
    cid = lax.axis_index("c")
    sid = lax.axis_index("s")
    wid = cid * NS + sid
    z16 = jnp.zeros((16,), jnp.float32)

    def zrow(i, carry):
        for l in range(D // 16):
            rows_v[i, pl.ds(l * 16, 16)] = z16
        return carry

    lax.fori_loop(0, K, zrow, 0)
    for r in range(RPT // K):
        pltpu.sync_copy(rows_v, acc_sh.at[pl.ds(sid * RPT + r * K, K)])
    plsc.subcore_barrier()

    total_chunks = src_hbm.shape[0] // K
    start, cnt = _chunk_range(wid, total_chunks)

    def body(i, carry):
        off = (start + i) * K
        pltpu.sync_copy(src_hbm.at[pl.ds(off, K)], src_v)
        pltpu.sync_copy(dst_hbm.at[pl.ds(off, K)], dst_v)
        pltpu.async_copy(y_hbm.at[src_v], rows_v, sem).wait()
        pltpu.sync_copy(rows_v, acc_sh.at[dst_v], add=True)
        return carry

    lax.fori_loop(0, cnt, body, 0)
    plsc.subcore_barrier()

    row0 = sid * RPT

    @pl.when(cid == 0)
    def _():
        pltpu.sync_copy(acc_sh.at[pl.ds(row0, RPT)], p0_hbm.at[pl.ds(row0, RPT)])

    @pl.when(cid == 1)
    def _():
        pltpu.sync_copy(acc_sh.at[pl.ds(row0, RPT)], p1_hbm.at[pl.ds(row0, RPT)])


def _agg_call(y, src, dst):
    f = pl.kernel(
        _agg_body,
        out_type=(
            jax.ShapeDtypeStruct((N_PAD, D), jnp.float32),
            jax.ShapeDtypeStruct((N_PAD, D), jnp.float32),
        ),
        mesh=_sc_mesh(),
        scratch_types=[
            pltpu.VMEM((K,), jnp.int32),
            pltpu.VMEM((K,), jnp.int32),
            pltpu.VMEM((K, D), jnp.float32),
            pltpu.VMEM_SHARED((N_PAD, D), jnp.float32),
            pltpu.SemaphoreType.DMA,
        ],
    )
    return f(y, src, dst)


def _dis_body(d0_ref, d1_ref, o_ref):
    o_ref[...] = lax.rsqrt(d0_ref[...] + d1_ref[...] + 1.0)


def _dis_call(d0, d1):
    return pl.pallas_call(
        _dis_body,
        out_shape=jax.ShapeDtypeStruct((N_PAD // D, D), jnp.float32),
    )(d0, d1)


def _lin_body(x_ref, w_ref, d_ref, o_ref):
    xw = jnp.dot(x_ref[...], w_ref[...], preferred_element_type=jnp.float32)
    o_ref[...] = xw * d_ref[...]


def _lin_call(xp, w, dis_col):
    return pl.pallas_call(
        _lin_body,
        grid=(N_PAD // RB,),
        in_specs=[
            pl.BlockSpec((RB, D), lambda i: (i, 0)),
            pl.BlockSpec((D, D), lambda i: (0, 0)),
            pl.BlockSpec((RB, 1), lambda i: (i, 0)),
        ],
        out_specs=pl.BlockSpec((RB, D), lambda i: (i, 0)),
        out_shape=jax.ShapeDtypeStruct((N_PAD, D), jnp.float32),
    )(xp, w, dis_col)


def _mid_body(a0_ref, a1_ref, y_ref, d_ref, b_ref, w_ref, o_ref):
    h = d_ref[...] * (a0_ref[...] + a1_ref[...] + y_ref[...]) + b_ref[...]
    h = jnp.maximum(h, 0.0)
    o_ref[...] = d_ref[...] * jnp.dot(
        h, w_ref[...], preferred_element_type=jnp.float32
    )


def _mid_call(a0, a1, y1, dis_col, b1, w2):
    return pl.pallas_call(
        _mid_body,
        grid=(N_PAD // RB,),
        in_specs=[
            pl.BlockSpec((RB, D), lambda i: (i, 0)),
            pl.BlockSpec((RB, D), lambda i: (i, 0)),
            pl.BlockSpec((RB, D), lambda i: (i, 0)),
            pl.BlockSpec((RB, 1), lambda i: (i, 0)),
            pl.BlockSpec((1, D), lambda i: (0, 0)),
            pl.BlockSpec((D, D), lambda i: (0, 0)),
        ],
        out_specs=pl.BlockSpec((RB, D), lambda i: (i, 0)),
        out_shape=jax.ShapeDtypeStruct((N_PAD, D), jnp.float32),
    )(a0, a1, y1, dis_col, b1, w2)


def _out_body(a0_ref, a1_ref, y_ref, d_ref, b_ref, o_ref):
    o_ref[...] = (
        d_ref[...] * (a0_ref[...] + a1_ref[...] + y_ref[...]) + b_ref[...]
    )


def _out_call(a0, a1, y2, dis_col, b2):
    return pl.pallas_call(
        _out_body,
        grid=(N_PAD // RB,),
        in_specs=[
            pl.BlockSpec((RB, D), lambda i: (i, 0)),
            pl.BlockSpec((RB, D), lambda i: (i, 0)),
            pl.BlockSpec((RB, D), lambda i: (i, 0)),
            pl.BlockSpec((RB, 1), lambda i: (i, 0)),
            pl.BlockSpec((1, D), lambda i: (0, 0)),
        ],
        out_specs=pl.BlockSpec((RB, D), lambda i: (i, 0)),
        out_shape=jax.ShapeDtypeStruct((N_PAD, D), jnp.float32),
    )(a0, a1, y2, dis_col, b2)


def kernel(x, edge_index, W1, b1, W2, b2):
    n, d_in = x.shape
    src = edge_index[0].astype(jnp.int32)
    dst = edge_index[1].astype(jnp.int32)
    xp = jnp.zeros((N_PAD, d_in), x.dtype).at[:n].set(x)

    deg0, deg1 = _deg_call(dst)
    dis = _dis_call(deg0.reshape(N_PAD // D, D), deg1.reshape(N_PAD // D, D))
    dis_col = dis.reshape(N_PAD, 1)

    y1 = _lin_call(xp, W1, dis_col)
    a0, a1 = _agg_call(y1, src, dst)
    y2 = _mid_call(a0, a1, y1, dis_col, b1.reshape(1, D), W2)
    o0, o1 = _agg_call(y2, src, dst)
    out = _out_call(o0, o1, y2, dis_col, b2.reshape(1, D))
    return out[:n]


# speedup vs baseline: 24.9141x; 1.4856x over previous
import functools

import jax
import jax.numpy as jnp
from jax import lax
from jax.experimental import pallas as pl
from jax.experimental.pallas import tpu as pltpu
from jax.experimental.pallas import tpu_sc as plsc

NC = 2
NS = 16
NW = NC * NS
D = 128
N_PAD = 10240
RPT = N_PAD // NS
K = 128
RB = 1000


def _sc_mesh():
    return plsc.VectorSubcoreMesh(
        core_axis_name="c", subcore_axis_name="s", num_cores=NC, num_subcores=NS
    )


def _chunk_range(wid, total_chunks):
    base = total_chunks // NW
    rem = total_chunks - base * NW
    start = wid * base + jnp.minimum(wid, rem)
    cnt = base + (wid < rem).astype(jnp.int32)
    return start, cnt


def _deg_body(dst_hbm, p0_hbm, p1_hbm, ones_v, dst_v, zbuf_v, deg_sh):
    cid = lax.axis_index("c")
    sid = lax.axis_index("s")
    wid = cid * NS + sid
    one16 = jnp.ones((16,), jnp.float32)
    z16 = jnp.zeros((16,), jnp.float32)
    for l in range(K // 16):
        ones_v[pl.ds(l * 16, 16)] = one16
        zbuf_v[pl.ds(l * 16, 16)] = z16
    for r in range(RPT // K):
        pltpu.sync_copy(zbuf_v, deg_sh.at[pl.ds(sid * RPT + r * K, K)])
    plsc.subcore_barrier()

    total_chunks = dst_hbm.shape[0] // K
    start, cnt = _chunk_range(wid, total_chunks)

    def body(i, carry):
        off = (start + i) * K
        pltpu.sync_copy(dst_hbm.at[pl.ds(off, K)], dst_v)
        pltpu.sync_copy(ones_v, deg_sh.at[dst_v], add=True)
        return carry

    lax.fori_loop(0, cnt, body, 0)
    plsc.subcore_barrier()

    row0 = sid * RPT

    @pl.when(cid == 0)
    def _():
        pltpu.sync_copy(deg_sh.at[pl.ds(row0, RPT)], p0_hbm.at[pl.ds(row0, RPT)])

    @pl.when(cid == 1)
    def _():
        pltpu.sync_copy(deg_sh.at[pl.ds(row0, RPT)], p1_hbm.at[pl.ds(row0, RPT)])


def _deg_call(dst):
    f = pl.kernel(
        _deg_body,
        out_type=(
            jax.ShapeDtypeStruct((N_PAD,), jnp.float32),
            jax.ShapeDtypeStruct((N_PAD,), jnp.float32),
        ),
        mesh=_sc_mesh(),
        scratch_types=[
            pltpu.VMEM((K,), jnp.float32),
            pltpu.VMEM((K,), jnp.int32),
            pltpu.VMEM((K,), jnp.float32),
            pltpu.VMEM_SHARED((N_PAD,), jnp.float32),
        ],
    )
    return f(dst)


def _agg_body(y_hbm, src_hbm, dst_hbm, p0_hbm, p1_hbm,
              src_v0, dst_v0, rows_v0, src_v1, dst_v1, rows_v1,
              acc_sh, sem0, sem1):
    cid = lax.axis_index("c")
    sid = lax.axis_index("s")
    wid = cid * NS + sid
    z16 = jnp.zeros((16,), jnp.float32)

    def zrow(i, carry):
        for l in range(D // 16):
            rows_v0[i, pl.ds(l * 16, 16)] = z16
        return carry

    lax.fori_loop(0, K, zrow, 0)
    for r in range(RPT // K):
        pltpu.sync_copy(rows_v0, acc_sh.at[pl.ds(sid * RPT + r * K, K)])
    plsc.subcore_barrier()

    total_chunks = src_hbm.shape[0] // K
    start, cnt = _chunk_range(wid, total_chunks)

    def load_idx(c, src_v, dst_v):
        off = (start + c) * K
        pltpu.sync_copy(src_hbm.at[pl.ds(off, K)], src_v)
        pltpu.sync_copy(dst_hbm.at[pl.ds(off, K)], dst_v)

    @pl.when(cnt > 0)
    def _():
        load_idx(0, src_v0, dst_v0)
        pltpu.async_copy(y_hbm.at[src_v0], rows_v0, sem0)

    def step(c, src_a, dst_a, rows_a, sem_a, src_b, dst_b, rows_b, sem_b):
        @pl.when(c + 1 < cnt)
        def _():
            load_idx(c + 1, src_b, dst_b)
            pltpu.async_copy(y_hbm.at[src_b], rows_b, sem_b)

        pltpu.make_async_copy(y_hbm.at[src_a], rows_a, sem_a).wait()
        pltpu.sync_copy(rows_a, acc_sh.at[dst_a], add=True)

    def pair_body(j, carry):
        c0 = 2 * j

        @pl.when(c0 < cnt)
        def _():
            step(c0, src_v0, dst_v0, rows_v0, sem0,
                 src_v1, dst_v1, rows_v1, sem1)

        @pl.when(c0 + 1 < cnt)
        def _():
            step(c0 + 1, src_v1, dst_v1, rows_v1, sem1,
                 src_v0, dst_v0, rows_v0, sem0)

        return carry

    lax.fori_loop(0, (cnt + 1) // 2, pair_body, 0)
    plsc.subcore_barrier()

    row0 = sid * RPT

    @pl.when(cid == 0)
    def _():
        pltpu.sync_copy(acc_sh.at[pl.ds(row0, RPT)], p0_hbm.at[pl.ds(row0, RPT)])

    @pl.when(cid == 1)
    def _():
        pltpu.sync_copy(acc_sh.at[pl.ds(row0, RPT)], p1_hbm.at[pl.ds(row0, RPT)])


def _agg_call(y, src, dst):
    f = pl.kernel(
        _agg_body,
        out_type=(
            jax.ShapeDtypeStruct((N_PAD, D), jnp.float32),
            jax.ShapeDtypeStruct((N_PAD, D), jnp.float32),
        ),
        mesh=_sc_mesh(),
        scratch_types=[
            pltpu.VMEM((K,), jnp.int32),
            pltpu.VMEM((K,), jnp.int32),
            pltpu.VMEM((K, D), jnp.float32),
            pltpu.VMEM((K,), jnp.int32),
            pltpu.VMEM((K,), jnp.int32),
            pltpu.VMEM((K, D), jnp.float32),
            pltpu.VMEM_SHARED((N_PAD, D), jnp.float32),
            pltpu.SemaphoreType.DMA,
            pltpu.SemaphoreType.DMA,
        ],
    )
    return f(y, src, dst)


def _dis_body(d0_ref, d1_ref, o_ref):
    o_ref[...] = lax.rsqrt(d0_ref[...] + d1_ref[...] + 1.0)


def _dis_call(d0, d1):
    return pl.pallas_call(
        _dis_body,
        out_shape=jax.ShapeDtypeStruct((N_PAD // D, D), jnp.float32),
    )(d0, d1)


def _lin_body(x_ref, w_ref, d_ref, o_ref):
    xw = jnp.dot(x_ref[...], w_ref[...], preferred_element_type=jnp.float32)
    o_ref[...] = xw * d_ref[...]


def _lin_call(xp, w, dis_col):
    return pl.pallas_call(
        _lin_body,
        grid=(10,),
        in_specs=[
            pl.BlockSpec((RB, D), lambda i: (i, 0)),
            pl.BlockSpec((D, D), lambda i: (0, 0)),
            pl.BlockSpec((RB, 1), lambda i: (i, 0)),
        ],
        out_specs=pl.BlockSpec((RB, D), lambda i: (i, 0)),
        out_shape=jax.ShapeDtypeStruct((10 * RB, D), jnp.float32),
    )(xp, w, dis_col)


def _mid_body(a0_ref, a1_ref, y_ref, d_ref, b_ref, w_ref, o_ref):
    h = d_ref[...] * (a0_ref[...] + a1_ref[...] + y_ref[...]) + b_ref[...]
    h = jnp.maximum(h, 0.0)
    o_ref[...] = d_ref[...] * jnp.dot(
        h, w_ref[...], preferred_element_type=jnp.float32
    )


def _mid_call(a0, a1, y1, dis_col, b1, w2):
    return pl.pallas_call(
        _mid_body,
        grid=(10,),
        in_specs=[
            pl.BlockSpec((RB, D), lambda i: (i, 0)),
            pl.BlockSpec((RB, D), lambda i: (i, 0)),
            pl.BlockSpec((RB, D), lambda i: (i, 0)),
            pl.BlockSpec((RB, 1), lambda i: (i, 0)),
            pl.BlockSpec((1, D), lambda i: (0, 0)),
            pl.BlockSpec((D, D), lambda i: (0, 0)),
        ],
        out_specs=pl.BlockSpec((RB, D), lambda i: (i, 0)),
        out_shape=jax.ShapeDtypeStruct((10 * RB, D), jnp.float32),
    )(a0, a1, y1, dis_col, b1, w2)


def _out_body(a0_ref, a1_ref, y_ref, d_ref, b_ref, o_ref):
    o_ref[...] = (
        d_ref[...] * (a0_ref[...] + a1_ref[...] + y_ref[...]) + b_ref[...]
    )


def _out_call(a0, a1, y2, dis_col, b2):
    return pl.pallas_call(
        _out_body,
        grid=(10,),
        in_specs=[
            pl.BlockSpec((RB, D), lambda i: (i, 0)),
            pl.BlockSpec((RB, D), lambda i: (i, 0)),
            pl.BlockSpec((RB, D), lambda i: (i, 0)),
            pl.BlockSpec((RB, 1), lambda i: (i, 0)),
            pl.BlockSpec((1, D), lambda i: (0, 0)),
        ],
        out_specs=pl.BlockSpec((RB, D), lambda i: (i, 0)),
        out_shape=jax.ShapeDtypeStruct((10 * RB, D), jnp.float32),
    )(a0, a1, y2, dis_col, b2)


def kernel(x, edge_index, W1, b1, W2, b2):
    n, d_in = x.shape
    src = edge_index[0].astype(jnp.int32)
    dst = edge_index[1].astype(jnp.int32)

    deg0, deg1 = _deg_call(dst)
    dis = _dis_call(deg0.reshape(N_PAD // D, D), deg1.reshape(N_PAD // D, D))
    dis_col = dis.reshape(N_PAD, 1)[:n]

    y1 = _lin_call(x, W1, dis_col)
    a0, a1 = _agg_call(y1, src, dst)
    y2 = _mid_call(a0, a1, y1, dis_col, b1.reshape(1, D), W2)
    o0, o1 = _agg_call(y2, src, dst)
    return _out_call(o0, o1, y2, dis_col, b2.reshape(1, D))
